# baseline (speedup 1.0000x reference)
import functools

import jax
import jax.numpy as jnp
from jax import lax
from jax.experimental import pallas as pl
from jax.experimental.pallas import tpu as pltpu
from jax.experimental.pallas import tpu_sc as plsc

N_WORKERS = 32
NBUF = 4
LOOK = 2
PADW = 128


def _make_gather(batch: int, hist: int, d_model: int):
    rows_per_w = batch // N_WORKERS
    n_sup = rows_per_w // NBUF
    mesh = plsc.VectorSubcoreMesh(core_axis_name="c", subcore_axis_name="s")

    @functools.partial(
        pl.kernel,
        mesh=mesh,
        out_type=jax.ShapeDtypeStruct((batch, hist, d_model), jnp.float32),
        compiler_params=pltpu.CompilerParams(use_tc_tiling_on_sc=False),
        scratch_types=[
            pltpu.VMEM((rows_per_w * hist,), jnp.int32),
            pltpu.VMEM((NBUF, hist, PADW), jnp.float32),
            pltpu.SemaphoreType.DMA((NBUF,)),
            pltpu.SemaphoreType.DMA((NBUF,)),
        ],
    )
    def gather_kernel(table_hbm, idx_hbm, out_hbm, idx_v, rows_v, gsem, ssem):
        wid = lax.axis_index("s") * 2 + lax.axis_index("c")
        base = wid * rows_per_w
        pltpu.sync_copy(
            idx_hbm.at[pl.ds(base * hist, rows_per_w * hist)], idx_v
        )

        def g_start(j, b):
            pltpu.async_copy(
                table_hbm.at[idx_v.at[pl.ds(j * hist, hist)]],
                rows_v.at[b],
                gsem.at[b],
            )

        def g_wait(b):
            pltpu.make_async_copy(
                table_hbm.at[idx_v.at[pl.ds(0, hist)]],
                rows_v.at[b],
                gsem.at[b],
            ).wait()

        def s_start(j, b):
            pltpu.async_copy(
                rows_v.at[b, :, pl.ds(0, d_model)],
                out_hbm.at[base + j],
                ssem.at[b],
            )

        def s_wait(b):
            pltpu.make_async_copy(
                rows_v.at[b, :, pl.ds(0, d_model)],
                out_hbm.at[base],
                ssem.at[b],
            ).wait()

        for b in range(LOOK):
            g_start(b, b)

        def super_body(s, carry):
            for b in range(NBUF):
                j = s * NBUF + b
                g_wait(b)
                s_start(j, b)
                jn = j + LOOK
                bn = (b + LOOK) % NBUF

                @pl.when(jn < rows_per_w)
                def _():
                    @pl.when(jn >= NBUF)
                    def _():
                        s_wait(bn)

                    g_start(jn, bn)

            return carry

        lax.fori_loop(0, n_sup, super_body, 0)

        for b in range(NBUF):
            s_wait(b)

    return gather_kernel


def kernel(input, table):
    b, h = input.shape
    v, d = table.shape
    table_p = jnp.pad(table, ((0, 0), (0, PADW - d)))
    return _make_gather(b, h, d)(table_p, input.reshape(-1))


# speedup vs baseline: 1.0140x; 1.0140x over previous
import functools

import jax
import jax.numpy as jnp
from jax import lax
from jax.experimental import pallas as pl
from jax.experimental.pallas import tpu as pltpu
from jax.experimental.pallas import tpu_sc as plsc

N_WORKERS = 32
NBUF = 8
LOOK = 4


def _make_gather(batch: int, hist: int, d_model: int):
    rows_per_w = batch // N_WORKERS
    n_sup = rows_per_w // NBUF
    mesh = plsc.VectorSubcoreMesh(core_axis_name="c", subcore_axis_name="s")

    @functools.partial(
        pl.kernel,
        mesh=mesh,
        out_type=jax.ShapeDtypeStruct((batch, hist, d_model), jnp.float32),
        compiler_params=pltpu.CompilerParams(use_tc_tiling_on_sc=False),
        scratch_types=[
            pltpu.VMEM((rows_per_w, hist), jnp.int32),
            pltpu.VMEM((NBUF, hist, d_model), jnp.float32),
            pltpu.SemaphoreType.DMA((NBUF,)),
            pltpu.SemaphoreType.DMA((NBUF,)),
        ],
    )
    def gather_kernel(table_hbm, idx_hbm, out_hbm, idx_v, rows_v, gsem, ssem):
        wid = lax.axis_index("s") * 2 + lax.axis_index("c")
        base = wid * rows_per_w
        pltpu.sync_copy(idx_hbm.at[pl.ds(base, rows_per_w)], idx_v)

        def g_start(j, b):
            pltpu.async_copy(
                table_hbm.at[idx_v.at[j]], rows_v.at[b], gsem.at[b]
            )

        def g_wait(b):
            pltpu.make_async_copy(
                table_hbm.at[idx_v.at[0]], rows_v.at[b], gsem.at[b]
            ).wait()

        def s_start(j, b):
            pltpu.async_copy(rows_v.at[b], out_hbm.at[base + j], ssem.at[b])

        def s_wait(b):
            pltpu.make_async_copy(
                rows_v.at[b], out_hbm.at[base], ssem.at[b]
            ).wait()

        for b in range(LOOK):
            g_start(b, b)

        def super_body(s, carry):
            for b in range(NBUF):
                j = s * NBUF + b
                g_wait(b)
                s_start(j, b)
                jn = j + LOOK
                bn = (b + LOOK) % NBUF

                @pl.when(jn < rows_per_w)
                def _():
                    @pl.when(jn >= NBUF)
                    def _():
                        s_wait(bn)

                    g_start(jn, bn)

            return carry

        lax.fori_loop(0, n_sup, super_body, 0)

        for b in range(NBUF):
            s_wait(b)

    return gather_kernel


def kernel(input, table):
    b, h = input.shape
    return _make_gather(b, h, table.shape[1])(table, input)
